# baseline (device time: 95817 ns/iter reference)
import jax
import jax.numpy as jnp
from jax import lax
from jax.experimental import pallas as pl
from jax.experimental.pallas import tpu as pltpu

T = 2048
D = 1024
TH = T // 2


def kernel(ids, E):
    v_local = E.shape[0]
    my_x = lax.axis_index("x")
    my_y = lax.axis_index("y")

    ids_half = lax.dynamic_slice(ids, (my_x * TH,), (TH,))
    local = ids_half - my_y * v_local
    local = jnp.where(local < 0, v_local, local)
    partial = jnp.take(E, local, axis=0, mode="fill", fill_value=0.0)
    partial = partial.astype(jnp.float32)

    C = 8
    CH = TH // C

    def body(partial_ref, out_ref, comm_ref, sum_ref, other_ref,
             send_a, recv_a, send_b, recv_b):
        x = lax.axis_index("x")
        y = lax.axis_index("y")
        y_nbr = (x, 1 - y)
        x_nbr = (1 - x, y)

        barrier = pltpu.get_barrier_semaphore()
        for nbr in (y_nbr, x_nbr):
            pl.semaphore_signal(
                barrier, inc=1, device_id=nbr,
                device_id_type=pl.DeviceIdType.MESH,
            )
        pl.semaphore_wait(barrier, 2)

        def rdma_a(c):
            return pltpu.make_async_remote_copy(
                src_ref=partial_ref.at[pl.ds(c * CH, CH), :],
                dst_ref=comm_ref.at[pl.ds(c * CH, CH), :],
                send_sem=send_a.at[c],
                recv_sem=recv_a.at[c],
                device_id=y_nbr,
                device_id_type=pl.DeviceIdType.MESH,
            )

        def rdma_b(c):
            return pltpu.make_async_remote_copy(
                src_ref=sum_ref.at[pl.ds(c * CH, CH), :],
                dst_ref=other_ref.at[pl.ds(c * CH, CH), :],
                send_sem=send_b.at[c],
                recv_sem=recv_b.at[c],
                device_id=x_nbr,
                device_id_type=pl.DeviceIdType.MESH,
            )

        for c in range(C):
            rdma_a(c).start()

        for c in range(C):
            rdma_a(c).wait_recv()
            sum_ref[pl.ds(c * CH, CH), :] = (
                partial_ref[pl.ds(c * CH, CH), :]
                + comm_ref[pl.ds(c * CH, CH), :]
            )
            rdma_b(c).start()

        for c in range(C):
            rdma_a(c).wait_send()
            rdma_b(c).wait()

        out_ref[pl.ds(x * TH, TH), :] = sum_ref[:, :]
        out_ref[pl.ds((1 - x) * TH, TH), :] = other_ref[:, :]

    return pl.pallas_call(
        body,
        out_shape=jax.ShapeDtypeStruct((T, D), jnp.float32),
        in_specs=[pl.BlockSpec(memory_space=pltpu.VMEM)],
        out_specs=pl.BlockSpec(memory_space=pltpu.VMEM),
        scratch_shapes=[
            pltpu.VMEM((TH, D), jnp.float32),
            pltpu.VMEM((TH, D), jnp.float32),
            pltpu.VMEM((TH, D), jnp.float32),
            pltpu.SemaphoreType.DMA((C,)),
            pltpu.SemaphoreType.DMA((C,)),
            pltpu.SemaphoreType.DMA((C,)),
            pltpu.SemaphoreType.DMA((C,)),
        ],
        compiler_params=pltpu.CompilerParams(collective_id=0),
    )(partial)


# device time: 85654 ns/iter; 1.1187x vs baseline; 1.1187x over previous
import jax
import jax.numpy as jnp
from jax import lax
from jax.experimental import pallas as pl
from jax.experimental.pallas import tpu as pltpu

T = 2048
D = 1024
TH = T // 2
C = 8
CH = TH // C


def kernel(ids, E):
    v_local = E.shape[0]
    my_x = lax.axis_index("x")
    my_y = lax.axis_index("y")

    ids_half = lax.dynamic_slice(ids, (my_x * TH,), (TH,))
    local = ids_half - my_y * v_local
    in_shard = (local >= 0) & (local < v_local)
    safe = jnp.clip(local, 0, v_local - 1).astype(jnp.int32)
    maskf = in_shard.astype(jnp.float32).reshape(TH, 1)

    def body(ids_ref, mask_ref, e_ref, out_ref,
             gath_ref, comm_ref, sum_ref, other_ref,
             row_sem, send_a, recv_a, send_b, recv_b):
        x = lax.axis_index("x")

        barrier = pltpu.get_barrier_semaphore()
        y_nbr = (x, 1 - lax.axis_index("y"))
        x_nbr = (1 - x, lax.axis_index("y"))
        for nbr in (y_nbr, x_nbr):
            pl.semaphore_signal(
                barrier, inc=1, device_id=nbr,
                device_id_type=pl.DeviceIdType.MESH,
            )
        pl.semaphore_wait(barrier, 2)

        def row_copy(t):
            row = ids_ref[t]
            return pltpu.make_async_copy(
                e_ref.at[pl.ds(row, 1), :],
                gath_ref.at[pl.ds(t, 1), :],
                row_sem,
            )

        def rdma_a(c):
            return pltpu.make_async_remote_copy(
                src_ref=gath_ref.at[pl.ds(c * CH, CH), :],
                dst_ref=comm_ref.at[pl.ds(c * CH, CH), :],
                send_sem=send_a.at[c],
                recv_sem=recv_a.at[c],
                device_id=y_nbr,
                device_id_type=pl.DeviceIdType.MESH,
            )

        def rdma_b(c):
            return pltpu.make_async_remote_copy(
                src_ref=sum_ref.at[pl.ds(c * CH, CH), :],
                dst_ref=other_ref.at[pl.ds(c * CH, CH), :],
                send_sem=send_b.at[c],
                recv_sem=recv_b.at[c],
                device_id=x_nbr,
                device_id_type=pl.DeviceIdType.MESH,
            )

        for c in range(C):
            lax.fori_loop(
                c * CH, (c + 1) * CH,
                lambda t, _: (row_copy(t).start(), 0)[1], 0,
            )
            lax.fori_loop(
                c * CH, (c + 1) * CH,
                lambda t, _: (row_copy(t).wait(), 0)[1], 0,
            )
            gath_ref[pl.ds(c * CH, CH), :] = (
                gath_ref[pl.ds(c * CH, CH), :]
                * mask_ref[pl.ds(c * CH, CH), :]
            )
            rdma_a(c).start()

        for c in range(C):
            rdma_a(c).wait_recv()
            sum_ref[pl.ds(c * CH, CH), :] = (
                gath_ref[pl.ds(c * CH, CH), :]
                + comm_ref[pl.ds(c * CH, CH), :]
            )
            rdma_b(c).start()

        for c in range(C):
            rdma_a(c).wait_send()
            rdma_b(c).wait()

        out_ref[pl.ds(x * TH, TH), :] = sum_ref[:, :]
        out_ref[pl.ds((1 - x) * TH, TH), :] = other_ref[:, :]

    return pl.pallas_call(
        body,
        out_shape=jax.ShapeDtypeStruct((T, D), jnp.float32),
        in_specs=[
            pl.BlockSpec(memory_space=pltpu.SMEM),
            pl.BlockSpec(memory_space=pltpu.VMEM),
            pl.BlockSpec(memory_space=pl.ANY),
        ],
        out_specs=pl.BlockSpec(memory_space=pltpu.VMEM),
        scratch_shapes=[
            pltpu.VMEM((TH, D), jnp.float32),
            pltpu.VMEM((TH, D), jnp.float32),
            pltpu.VMEM((TH, D), jnp.float32),
            pltpu.VMEM((TH, D), jnp.float32),
            pltpu.SemaphoreType.DMA,
            pltpu.SemaphoreType.DMA((C,)),
            pltpu.SemaphoreType.DMA((C,)),
            pltpu.SemaphoreType.DMA((C,)),
            pltpu.SemaphoreType.DMA((C,)),
        ],
        compiler_params=pltpu.CompilerParams(collective_id=0),
    )(safe, maskf, E)


# device time: 84469 ns/iter; 1.1343x vs baseline; 1.0140x over previous
import jax
import jax.numpy as jnp
from jax import lax
from jax.experimental import pallas as pl
from jax.experimental.pallas import tpu as pltpu

T = 2048
D = 1024
TH = T // 2
C = 8
CH = TH // C


def kernel(ids, E):
    v_local = E.shape[0]
    my_x = lax.axis_index("x")
    my_y = lax.axis_index("y")

    ids_half = lax.dynamic_slice(ids, (my_x * TH,), (TH,))
    local = ids_half - my_y * v_local
    in_shard = (local >= 0) & (local < v_local)
    safe = jnp.clip(local, 0, v_local - 1).astype(jnp.int32)
    maskf = in_shard.astype(jnp.float32).reshape(TH, 1)

    def body(ids_ref, mask_ref, e_ref, out_ref,
             gath_ref, comm_ref, sum_ref, other_ref,
             row_sem, send_a, recv_a, send_b, recv_b):
        x = lax.axis_index("x")

        barrier = pltpu.get_barrier_semaphore()
        y_nbr = (x, 1 - lax.axis_index("y"))
        x_nbr = (1 - x, lax.axis_index("y"))
        for nbr in (y_nbr, x_nbr):
            pl.semaphore_signal(
                barrier, inc=1, device_id=nbr,
                device_id_type=pl.DeviceIdType.MESH,
            )
        pl.semaphore_wait(barrier, 2)

        def row_copy(t, c):
            row = ids_ref[t]
            return pltpu.make_async_copy(
                e_ref.at[pl.ds(row, 1), :],
                gath_ref.at[pl.ds(t, 1), :],
                row_sem.at[c],
            )

        def rdma_a(c):
            return pltpu.make_async_remote_copy(
                src_ref=gath_ref.at[pl.ds(c * CH, CH), :],
                dst_ref=comm_ref.at[pl.ds(c * CH, CH), :],
                send_sem=send_a.at[c],
                recv_sem=recv_a.at[c],
                device_id=y_nbr,
                device_id_type=pl.DeviceIdType.MESH,
            )

        def rdma_b(c):
            return pltpu.make_async_remote_copy(
                src_ref=sum_ref.at[pl.ds(c * CH, CH), :],
                dst_ref=other_ref.at[pl.ds(c * CH, CH), :],
                send_sem=send_b.at[c],
                recv_sem=recv_b.at[c],
                device_id=x_nbr,
                device_id_type=pl.DeviceIdType.MESH,
            )

        for c in range(C):
            lax.fori_loop(
                c * CH, (c + 1) * CH,
                lambda t, _, c=c: (row_copy(t, c).start(), 0)[1], 0,
            )

        for c in range(C):
            lax.fori_loop(
                c * CH, (c + 1) * CH,
                lambda t, _, c=c: (row_copy(t, c).wait(), 0)[1], 0,
            )
            gath_ref[pl.ds(c * CH, CH), :] = (
                gath_ref[pl.ds(c * CH, CH), :]
                * mask_ref[pl.ds(c * CH, CH), :]
            )
            rdma_a(c).start()

        for c in range(C):
            rdma_a(c).wait_recv()
            sum_ref[pl.ds(c * CH, CH), :] = (
                gath_ref[pl.ds(c * CH, CH), :]
                + comm_ref[pl.ds(c * CH, CH), :]
            )
            rdma_b(c).start()

        for c in range(C):
            rdma_a(c).wait_send()
            rdma_b(c).wait()

        out_ref[pl.ds(x * TH, TH), :] = sum_ref[:, :]
        out_ref[pl.ds((1 - x) * TH, TH), :] = other_ref[:, :]

    return pl.pallas_call(
        body,
        out_shape=jax.ShapeDtypeStruct((T, D), jnp.float32),
        in_specs=[
            pl.BlockSpec(memory_space=pltpu.SMEM),
            pl.BlockSpec(memory_space=pltpu.VMEM),
            pl.BlockSpec(memory_space=pl.ANY),
        ],
        out_specs=pl.BlockSpec(memory_space=pltpu.VMEM),
        scratch_shapes=[
            pltpu.VMEM((TH, D), jnp.float32),
            pltpu.VMEM((TH, D), jnp.float32),
            pltpu.VMEM((TH, D), jnp.float32),
            pltpu.VMEM((TH, D), jnp.float32),
            pltpu.SemaphoreType.DMA((C,)),
            pltpu.SemaphoreType.DMA((C,)),
            pltpu.SemaphoreType.DMA((C,)),
            pltpu.SemaphoreType.DMA((C,)),
            pltpu.SemaphoreType.DMA((C,)),
        ],
        compiler_params=pltpu.CompilerParams(collective_id=0),
    )(safe, maskf, E)


# device time: 81667 ns/iter; 1.1733x vs baseline; 1.0343x over previous
import jax
import jax.numpy as jnp
from jax import lax
from jax.experimental import pallas as pl
from jax.experimental.pallas import tpu as pltpu

T = 2048
D = 1024
TH = T // 2
C = 8
CH = TH // C
ROW_BYTES = D * 4


def kernel(ids, E):
    v_local = E.shape[0]
    my_x = lax.axis_index("x")
    my_y = lax.axis_index("y")

    ids_half = lax.dynamic_slice(ids, (my_x * TH,), (TH,))
    local = ids_half - my_y * v_local
    owned = (local >= 0) & (local < v_local)
    t_idx = jnp.arange(TH, dtype=jnp.int32)
    perm = jnp.argsort(jnp.where(owned, t_idx, TH + t_idx)).astype(jnp.int32)
    src_rows = jnp.clip(local, 0, v_local - 1).astype(jnp.int32)[perm]
    dst_slots = perm
    cnt = jnp.sum(owned.reshape(C, CH).astype(jnp.int32), axis=1)
    offs = jnp.concatenate(
        [jnp.zeros((1,), jnp.int32), jnp.cumsum(cnt).astype(jnp.int32)]
    )

    def body(src_ref, dst_ref, offs_ref, e_ref, out_ref,
             gath_ref, comm_ref, sum_ref, other_ref,
             row_sem, send_a, recv_a, send_b, recv_b):
        x = lax.axis_index("x")
        y = lax.axis_index("y")
        y_nbr = (x, 1 - y)
        x_nbr = (1 - x, y)

        barrier = pltpu.get_barrier_semaphore()
        for nbr in (y_nbr, x_nbr):
            pl.semaphore_signal(
                barrier, inc=1, device_id=nbr,
                device_id_type=pl.DeviceIdType.MESH,
            )
        pl.semaphore_wait(barrier, 2)

        gath_ref[:, :] = jnp.zeros((TH, D), jnp.float32)

        def issue_chunk(c):
            def one(k, _):
                pltpu.make_async_copy(
                    e_ref.at[pl.ds(src_ref[k], 1), :],
                    gath_ref.at[pl.ds(dst_ref[k], 1), :],
                    row_sem.at[c],
                ).start()
                return 0

            lax.fori_loop(offs_ref[c], offs_ref[c + 1], one, 0)

        def wait_chunk(c):
            def one(k, _):
                pltpu.make_async_copy(
                    e_ref.at[pl.ds(0, 1), :],
                    gath_ref.at[pl.ds(c * CH, 1), :],
                    row_sem.at[c],
                ).wait()
                return 0

            lax.fori_loop(offs_ref[c], offs_ref[c + 1], one, 0)

        def rdma_a(c):
            return pltpu.make_async_remote_copy(
                src_ref=gath_ref.at[pl.ds(c * CH, CH), :],
                dst_ref=comm_ref.at[pl.ds(c * CH, CH), :],
                send_sem=send_a.at[c],
                recv_sem=recv_a.at[c],
                device_id=y_nbr,
                device_id_type=pl.DeviceIdType.MESH,
            )

        def rdma_b(c):
            return pltpu.make_async_remote_copy(
                src_ref=sum_ref.at[pl.ds(c * CH, CH), :],
                dst_ref=other_ref.at[pl.ds(c * CH, CH), :],
                send_sem=send_b.at[c],
                recv_sem=recv_b.at[c],
                device_id=x_nbr,
                device_id_type=pl.DeviceIdType.MESH,
            )

        issue_chunk(0)
        issue_chunk(1)
        for c in range(C):
            wait_chunk(c)
            rdma_a(c).start()
            if c + 2 < C:
                issue_chunk(c + 2)

        for c in range(C):
            rdma_a(c).wait_recv()
            sum_ref[pl.ds(c * CH, CH), :] = (
                gath_ref[pl.ds(c * CH, CH), :]
                + comm_ref[pl.ds(c * CH, CH), :]
            )
            rdma_b(c).start()

        for c in range(C):
            rdma_a(c).wait_send()
            rdma_b(c).wait()

        out_ref[pl.ds(x * TH, TH), :] = sum_ref[:, :]
        out_ref[pl.ds((1 - x) * TH, TH), :] = other_ref[:, :]

    return pl.pallas_call(
        body,
        out_shape=jax.ShapeDtypeStruct((T, D), jnp.float32),
        in_specs=[
            pl.BlockSpec(memory_space=pltpu.SMEM),
            pl.BlockSpec(memory_space=pltpu.SMEM),
            pl.BlockSpec(memory_space=pltpu.SMEM),
            pl.BlockSpec(memory_space=pl.ANY),
        ],
        out_specs=pl.BlockSpec(memory_space=pltpu.VMEM),
        scratch_shapes=[
            pltpu.VMEM((TH, D), jnp.float32),
            pltpu.VMEM((TH, D), jnp.float32),
            pltpu.VMEM((TH, D), jnp.float32),
            pltpu.VMEM((TH, D), jnp.float32),
            pltpu.SemaphoreType.DMA((C,)),
            pltpu.SemaphoreType.DMA((C,)),
            pltpu.SemaphoreType.DMA((C,)),
            pltpu.SemaphoreType.DMA((C,)),
            pltpu.SemaphoreType.DMA((C,)),
        ],
        compiler_params=pltpu.CompilerParams(collective_id=0),
    )(src_rows, dst_slots, offs, E)


# device time: 77972 ns/iter; 1.2289x vs baseline; 1.0474x over previous
import jax
import jax.numpy as jnp
from jax import lax
from jax.experimental import pallas as pl
from jax.experimental.pallas import tpu as pltpu

T = 2048
D = 1024
TH = T // 2
C = 8
CH = TH // C


def kernel(ids, E):
    v_local = E.shape[0]
    my_x = lax.axis_index("x")
    my_y = lax.axis_index("y")

    ids_half = lax.dynamic_slice(ids, (my_x * TH,), (TH,))
    local = ids_half - my_y * v_local
    in_shard = (local >= 0) & (local < v_local)
    safe = jnp.clip(local, 0, v_local - 1).astype(jnp.int32)
    maskf = in_shard.astype(jnp.float32).reshape(TH, 1)

    def body(ids_ref, mask_ref, e_ref, out_ref,
             gath_ref, comm_ref, sum_ref, other_ref,
             row_sem, send_a, recv_a, send_b, recv_b):
        x = lax.axis_index("x")
        y = lax.axis_index("y")
        y_nbr = (x, 1 - y)
        x_nbr = (1 - x, y)

        barrier = pltpu.get_barrier_semaphore()
        for nbr in (y_nbr, x_nbr):
            pl.semaphore_signal(
                barrier, inc=1, device_id=nbr,
                device_id_type=pl.DeviceIdType.MESH,
            )
        pl.semaphore_wait(barrier, 2)

        def issue_chunk(c):
            def one(t, _):
                pltpu.make_async_copy(
                    e_ref.at[pl.ds(ids_ref[t], 1), :],
                    gath_ref.at[pl.ds(t, 1), :],
                    row_sem.at[c],
                ).start()
                return 0

            lax.fori_loop(c * CH, (c + 1) * CH, one, 0)

        def wait_chunk(c):
            pltpu.make_async_copy(
                e_ref.at[pl.ds(0, CH), :],
                gath_ref.at[pl.ds(c * CH, CH), :],
                row_sem.at[c],
            ).wait()

        def rdma_a(c):
            return pltpu.make_async_remote_copy(
                src_ref=gath_ref.at[pl.ds(c * CH, CH), :],
                dst_ref=comm_ref.at[pl.ds(c * CH, CH), :],
                send_sem=send_a.at[c],
                recv_sem=recv_a.at[c],
                device_id=y_nbr,
                device_id_type=pl.DeviceIdType.MESH,
            )

        def rdma_b(c):
            return pltpu.make_async_remote_copy(
                src_ref=sum_ref.at[pl.ds(c * CH, CH), :],
                dst_ref=other_ref.at[pl.ds(c * CH, CH), :],
                send_sem=send_b.at[c],
                recv_sem=recv_b.at[c],
                device_id=x_nbr,
                device_id_type=pl.DeviceIdType.MESH,
            )

        issue_chunk(0)
        issue_chunk(1)
        for c in range(C):
            wait_chunk(c)
            gath_ref[pl.ds(c * CH, CH), :] = (
                gath_ref[pl.ds(c * CH, CH), :]
                * mask_ref[pl.ds(c * CH, CH), :]
            )
            rdma_a(c).start()
            if c + 2 < C:
                issue_chunk(c + 2)

        for c in range(C):
            rdma_a(c).wait_recv()
            sum_ref[pl.ds(c * CH, CH), :] = (
                gath_ref[pl.ds(c * CH, CH), :]
                + comm_ref[pl.ds(c * CH, CH), :]
            )
            rdma_b(c).start()

        for c in range(C):
            rdma_a(c).wait_send()
            rdma_b(c).wait()

        out_ref[pl.ds(x * TH, TH), :] = sum_ref[:, :]
        out_ref[pl.ds((1 - x) * TH, TH), :] = other_ref[:, :]

    return pl.pallas_call(
        body,
        out_shape=jax.ShapeDtypeStruct((T, D), jnp.float32),
        in_specs=[
            pl.BlockSpec(memory_space=pltpu.SMEM),
            pl.BlockSpec(memory_space=pltpu.VMEM),
            pl.BlockSpec(memory_space=pl.ANY),
        ],
        out_specs=pl.BlockSpec(memory_space=pltpu.VMEM),
        scratch_shapes=[
            pltpu.VMEM((TH, D), jnp.float32),
            pltpu.VMEM((TH, D), jnp.float32),
            pltpu.VMEM((TH, D), jnp.float32),
            pltpu.VMEM((TH, D), jnp.float32),
            pltpu.SemaphoreType.DMA((C,)),
            pltpu.SemaphoreType.DMA((C,)),
            pltpu.SemaphoreType.DMA((C,)),
            pltpu.SemaphoreType.DMA((C,)),
            pltpu.SemaphoreType.DMA((C,)),
        ],
        compiler_params=pltpu.CompilerParams(collective_id=0),
    )(safe, maskf, E)


# device time: 77834 ns/iter; 1.2310x vs baseline; 1.0018x over previous
import jax
import jax.numpy as jnp
from jax import lax
from jax.experimental import pallas as pl
from jax.experimental.pallas import tpu as pltpu

T = 2048
D = 1024
TH = T // 2
C = 8
CH = TH // C


def kernel(ids, E):
    v_local = E.shape[0]
    my_x = lax.axis_index("x")
    my_y = lax.axis_index("y")

    ids_half = lax.dynamic_slice(ids, (my_x * TH,), (TH,))
    local = ids_half - my_y * v_local
    in_shard = (local >= 0) & (local < v_local)
    safe = jnp.clip(local, 0, v_local - 1).astype(jnp.int32)
    maskf = in_shard.astype(jnp.float32).reshape(TH, 1)

    def body(ids_ref, mask_ref, e_ref, out_ref,
             gath_ref, comm_ref, sum_ref, other_ref,
             row_sem, send_a, recv_a, send_b, recv_b):
        x = lax.axis_index("x")
        y = lax.axis_index("y")
        y_nbr = (x, 1 - y)
        x_nbr = (1 - x, y)

        barrier = pltpu.get_barrier_semaphore()
        for nbr in (y_nbr, x_nbr):
            pl.semaphore_signal(
                barrier, inc=1, device_id=nbr,
                device_id_type=pl.DeviceIdType.MESH,
            )
        pl.semaphore_wait(barrier, 2)

        def issue_chunk(c):
            def one(t, _):
                pltpu.make_async_copy(
                    e_ref.at[pl.ds(ids_ref[t], 1), :],
                    gath_ref.at[pl.ds(t, 1), :],
                    row_sem.at[c],
                ).start()
                return 0

            lax.fori_loop(c * CH, (c + 1) * CH, one, 0)

        def wait_chunk(c):
            pltpu.make_async_copy(
                e_ref.at[pl.ds(0, CH), :],
                gath_ref.at[pl.ds(c * CH, CH), :],
                row_sem.at[c],
            ).wait()

        def rdma_a(c):
            return pltpu.make_async_remote_copy(
                src_ref=gath_ref.at[pl.ds(c * CH, CH), :],
                dst_ref=comm_ref.at[pl.ds(c * CH, CH), :],
                send_sem=send_a.at[c],
                recv_sem=recv_a.at[c],
                device_id=y_nbr,
                device_id_type=pl.DeviceIdType.MESH,
            )

        def rdma_b(c):
            return pltpu.make_async_remote_copy(
                src_ref=sum_ref.at[pl.ds(c * CH, CH), :],
                dst_ref=other_ref.at[pl.ds(c * CH, CH), :],
                send_sem=send_b.at[c],
                recv_sem=recv_b.at[c],
                device_id=x_nbr,
                device_id_type=pl.DeviceIdType.MESH,
            )

        issue_chunk(0)
        issue_chunk(1)
        for c in range(C):
            if c + 2 < C:
                issue_chunk(c + 2)
            wait_chunk(c)
            rdma_a(c).start()

        for c in range(C):
            rdma_a(c).wait_recv()
            sum_ref[pl.ds(c * CH, CH), :] = jnp.where(
                mask_ref[pl.ds(c * CH, CH), :] > 0,
                gath_ref[pl.ds(c * CH, CH), :],
                comm_ref[pl.ds(c * CH, CH), :],
            )
            rdma_b(c).start()

        for c in range(C):
            rdma_a(c).wait_send()
            rdma_b(c).wait()

        out_ref[pl.ds(x * TH, TH), :] = sum_ref[:, :]
        out_ref[pl.ds((1 - x) * TH, TH), :] = other_ref[:, :]

    return pl.pallas_call(
        body,
        out_shape=jax.ShapeDtypeStruct((T, D), jnp.float32),
        in_specs=[
            pl.BlockSpec(memory_space=pltpu.SMEM),
            pl.BlockSpec(memory_space=pltpu.VMEM),
            pl.BlockSpec(memory_space=pl.ANY),
        ],
        out_specs=pl.BlockSpec(memory_space=pltpu.VMEM),
        scratch_shapes=[
            pltpu.VMEM((TH, D), jnp.float32),
            pltpu.VMEM((TH, D), jnp.float32),
            pltpu.VMEM((TH, D), jnp.float32),
            pltpu.VMEM((TH, D), jnp.float32),
            pltpu.SemaphoreType.DMA((C,)),
            pltpu.SemaphoreType.DMA((C,)),
            pltpu.SemaphoreType.DMA((C,)),
            pltpu.SemaphoreType.DMA((C,)),
            pltpu.SemaphoreType.DMA((C,)),
        ],
        compiler_params=pltpu.CompilerParams(collective_id=0),
    )(safe, maskf, E)


# device time: 77502 ns/iter; 1.2363x vs baseline; 1.0043x over previous
import jax
import jax.numpy as jnp
from jax import lax
from jax.experimental import pallas as pl
from jax.experimental.pallas import tpu as pltpu

T = 2048
D = 1024
TH = T // 2
SIZES = (64, 128, 160, 160, 160, 160, 128, 64)
C = len(SIZES)
OFFS = tuple(sum(SIZES[:c]) for c in range(C))
assert sum(SIZES) == TH


def kernel(ids, E):
    v_local = E.shape[0]
    my_x = lax.axis_index("x")
    my_y = lax.axis_index("y")

    ids_half = lax.dynamic_slice(ids, (my_x * TH,), (TH,))
    local = ids_half - my_y * v_local
    in_shard = (local >= 0) & (local < v_local)
    safe = jnp.clip(local, 0, v_local - 1).astype(jnp.int32)
    maskf = in_shard.astype(jnp.float32).reshape(TH, 1)

    def body(ids_ref, mask_ref, e_ref, out_ref,
             gath_ref, comm_ref, sum_ref, other_ref,
             row_sem, send_a, recv_a, send_b, recv_b):
        x = lax.axis_index("x")
        y = lax.axis_index("y")
        y_nbr = (x, 1 - y)
        x_nbr = (1 - x, y)

        barrier = pltpu.get_barrier_semaphore()
        for nbr in (y_nbr, x_nbr):
            pl.semaphore_signal(
                barrier, inc=1, device_id=nbr,
                device_id_type=pl.DeviceIdType.MESH,
            )
        pl.semaphore_wait(barrier, 2)

        def issue_chunk(c):
            def one(t, _):
                pltpu.make_async_copy(
                    e_ref.at[pl.ds(ids_ref[t], 1), :],
                    gath_ref.at[pl.ds(t, 1), :],
                    row_sem.at[c],
                ).start()
                return 0

            lax.fori_loop(OFFS[c], OFFS[c] + SIZES[c], one, 0)

        def wait_chunk(c):
            pltpu.make_async_copy(
                e_ref.at[pl.ds(0, SIZES[c]), :],
                gath_ref.at[pl.ds(OFFS[c], SIZES[c]), :],
                row_sem.at[c],
            ).wait()

        def rdma_a(c):
            return pltpu.make_async_remote_copy(
                src_ref=gath_ref.at[pl.ds(OFFS[c], SIZES[c]), :],
                dst_ref=comm_ref.at[pl.ds(OFFS[c], SIZES[c]), :],
                send_sem=send_a.at[c],
                recv_sem=recv_a.at[c],
                device_id=y_nbr,
                device_id_type=pl.DeviceIdType.MESH,
            )

        def rdma_b(c):
            return pltpu.make_async_remote_copy(
                src_ref=sum_ref.at[pl.ds(OFFS[c], SIZES[c]), :],
                dst_ref=other_ref.at[pl.ds(OFFS[c], SIZES[c]), :],
                send_sem=send_b.at[c],
                recv_sem=recv_b.at[c],
                device_id=x_nbr,
                device_id_type=pl.DeviceIdType.MESH,
            )

        issue_chunk(0)
        issue_chunk(1)
        for c in range(C):
            if c + 2 < C:
                issue_chunk(c + 2)
            wait_chunk(c)
            rdma_a(c).start()

        for c in range(C):
            rdma_a(c).wait_recv()
            sum_ref[pl.ds(OFFS[c], SIZES[c]), :] = jnp.where(
                mask_ref[pl.ds(OFFS[c], SIZES[c]), :] > 0,
                gath_ref[pl.ds(OFFS[c], SIZES[c]), :],
                comm_ref[pl.ds(OFFS[c], SIZES[c]), :],
            )
            rdma_b(c).start()
            out_ref[pl.ds(x * TH + OFFS[c], SIZES[c]), :] = sum_ref[
                pl.ds(OFFS[c], SIZES[c]), :
            ]

        for c in range(C):
            rdma_a(c).wait_send()
            rdma_b(c).wait()

        out_ref[pl.ds((1 - x) * TH, TH), :] = other_ref[:, :]

    return pl.pallas_call(
        body,
        out_shape=jax.ShapeDtypeStruct((T, D), jnp.float32),
        in_specs=[
            pl.BlockSpec(memory_space=pltpu.SMEM),
            pl.BlockSpec(memory_space=pltpu.VMEM),
            pl.BlockSpec(memory_space=pl.ANY),
        ],
        out_specs=pl.BlockSpec(memory_space=pltpu.VMEM),
        scratch_shapes=[
            pltpu.VMEM((TH, D), jnp.float32),
            pltpu.VMEM((TH, D), jnp.float32),
            pltpu.VMEM((TH, D), jnp.float32),
            pltpu.VMEM((TH, D), jnp.float32),
            pltpu.SemaphoreType.DMA((C,)),
            pltpu.SemaphoreType.DMA((C,)),
            pltpu.SemaphoreType.DMA((C,)),
            pltpu.SemaphoreType.DMA((C,)),
            pltpu.SemaphoreType.DMA((C,)),
        ],
        compiler_params=pltpu.CompilerParams(collective_id=0),
    )(safe, maskf, E)
